# baseline (speedup 1.0000x reference)
import jax
import jax.numpy as jnp
from jax.experimental import pallas as pl

BB_ = 16
N1 = 2048
NSUB = 1024
EMB = 512
HID = 256
_EPS = 1e-5
_CNT = BB_ * N1
_NT = 2
_TN = N1 // _NT


def _lrelu(x):
    return jnp.where(x >= 0, x, x * 0.01)


def _mm(a, b):
    return jax.lax.dot_general(a, b, (((1,), (0,)), ((), ())),
                               preferred_element_type=jnp.float32)


def _k1a(src_ref, tgt_ref, W1_ref, be1_ref, W2_ref, be2_ref,
         W1t_ref, bt1_ref, W2t_ref, bt2_ref, se_ref, te_ref):
    sf = _lrelu(_mm(W1_ref[...], src_ref[0]) + be1_ref[...])
    se_ref[0] = _mm(W2_ref[...], sf) + be2_ref[...]
    ia = jnp.max(sf, axis=1, keepdims=True)
    tf = _lrelu(_mm(W1t_ref[...], tgt_ref[0]) + bt1_ref[...])
    te_ref[0] = _mm(W2t_ref[...], tf + ia) + bt2_ref[...]


def _k1b(se_ref, te_ref, MW1_ref, Mb1_ref, pre1_ref, s1_ref):
    b = pl.program_id(0)
    t = pl.program_id(1)
    se = se_ref[0]
    te = te_ref[0]
    sinv = 1.0 / jnp.sqrt(jnp.sum(se * se, axis=0, keepdims=True))
    tinv = 1.0 / jnp.sqrt(jnp.sum(te * te, axis=0, keepdims=True))
    sn = se * sinv
    tn = te * tinv
    cos = jax.lax.dot_general(tn, sn, (((0,), (0,)), ((), ())),
                              preferred_element_type=jnp.float32)
    pre1 = _mm(MW1_ref[...], cos) + Mb1_ref[...]
    pre1_ref[0] = pre1
    s = jnp.sum(pre1, axis=1)[None, :]

    @pl.when((b == 0) & (t == 0))
    def _():
        s1_ref[...] = s

    @pl.when((b != 0) | (t != 0))
    def _():
        s1_ref[...] += s


def _kvar(pre_ref, m_ref, q_ref):
    b = pl.program_id(0)
    d = pre_ref[0] - m_ref[...]
    q = jnp.sum(d * d, axis=1)[None, :]

    @pl.when(b == 0)
    def _():
        q_ref[...] = q

    @pl.when(b != 0)
    def _():
        q_ref[...] += q


def _klayer(pre_ref, m_ref, r_ref, bb_ref, W_ref, b_ref, out_ref, s_ref):
    i = pl.program_id(0)
    first = i == 0
    if pre_ref.shape[2] != N1:
        t = pl.program_id(1)
        first = first & (t == 0)
    h = _lrelu((pre_ref[0] - m_ref[...]) * r_ref[...] + bb_ref[...])
    pre = _mm(W_ref[...], h) + b_ref[...]
    out_ref[0] = pre
    s = jnp.sum(pre, axis=1)[None, :]

    @pl.when(first)
    def _():
        s_ref[...] = s

    @pl.when(jnp.logical_not(first))
    def _():
        s_ref[...] += s


def _k5(pre_ref, m_ref, r_ref, bb_ref, W5_ref, b5_ref, mask_ref, oh_ref):
    h = _lrelu((pre_ref[0] - m_ref[...]) * r_ref[...] + bb_ref[...])
    pre5 = _mm(W5_ref[...], h) + b5_ref[...]
    m = jax.nn.sigmoid(pre5)
    mask_ref[0] = m
    vt = jnp.transpose(m)
    col = jax.lax.broadcasted_iota(jnp.int32, (N1, N1), 1)
    row = jax.lax.broadcasted_iota(jnp.int32, (N1, N1), 0)
    greater = (vt > m).astype(jnp.int32)
    eq_lower = ((vt == m) & (row < col)).astype(jnp.int32)
    rank = jnp.sum(greater + eq_lower, axis=0, keepdims=True)
    oh_ref[0] = (rank < NSUB).astype(jnp.int32)


def _full(shape):
    nd = len(shape)
    return pl.BlockSpec(shape, lambda *_: (0,) * nd)


def _var_pass(pre, m, cin):
    q = pl.pallas_call(
        _kvar,
        grid=(BB_,),
        in_specs=[pl.BlockSpec((1, cin, N1), lambda b: (b, 0, 0)),
                  _full((cin, 1))],
        out_specs=_full((1, cin)),
        out_shape=jax.ShapeDtypeStruct((1, cin), jnp.float32),
    )(pre, m)
    return q[0] / _CNT


def kernel(src, tgt, W1, be1, W2, be2, W1t, bt1, W2t, bt2,
           MW1, Mb1, MW2, Mb2, MW3, Mb3, MW4, Mb4, MW5, Mb5,
           BG1, BB1, BG2, BB2, BG3, BB3, BG4, BB4):
    f32 = jnp.float32
    se, te = pl.pallas_call(
        _k1a,
        grid=(BB_,),
        in_specs=[
            pl.BlockSpec((1, 3, N1), lambda b: (b, 0, 0)),
            pl.BlockSpec((1, 3, NSUB), lambda b: (b, 0, 0)),
            _full((HID, 3)), _full((HID, 1)),
            _full((EMB, HID)), _full((EMB, 1)),
            _full((HID, 3)), _full((HID, 1)),
            _full((EMB, HID)), _full((EMB, 1)),
        ],
        out_specs=[
            pl.BlockSpec((1, EMB, N1), lambda b: (b, 0, 0)),
            pl.BlockSpec((1, EMB, NSUB), lambda b: (b, 0, 0)),
        ],
        out_shape=[
            jax.ShapeDtypeStruct((BB_, EMB, N1), f32),
            jax.ShapeDtypeStruct((BB_, EMB, NSUB), f32),
        ],
    )(src, tgt, W1, be1[:, None], W2, be2[:, None],
      W1t, bt1[:, None], W2t, bt2[:, None])

    pre1, s1 = pl.pallas_call(
        _k1b,
        grid=(BB_, _NT),
        in_specs=[
            pl.BlockSpec((1, EMB, _TN), lambda b, t: (b, 0, t)),
            pl.BlockSpec((1, EMB, NSUB), lambda b, t: (b, 0, 0)),
            _full((NSUB, NSUB)), _full((NSUB, 1)),
        ],
        out_specs=[
            pl.BlockSpec((1, NSUB, _TN), lambda b, t: (b, 0, t)),
            _full((1, NSUB)),
        ],
        out_shape=[
            jax.ShapeDtypeStruct((BB_, NSUB, N1), f32),
            jax.ShapeDtypeStruct((1, NSUB), f32),
        ],
    )(se, te, MW1, Mb1[:, None])

    def layer(pre, s, g, bb, W, wb, cout, nt):
        cin = pre.shape[1]
        tn = N1 // nt
        m = s[0] / _CNT
        v = _var_pass(pre, m[:, None], cin)
        r = g / jnp.sqrt(v + _EPS)
        grid = (BB_, nt) if nt > 1 else (BB_,)
        if nt > 1:
            pmap = lambda b, t: (b, 0, t)
        else:
            pmap = lambda b: (b, 0, 0)
        return pl.pallas_call(
            _klayer,
            grid=grid,
            in_specs=[
                pl.BlockSpec((1, cin, tn), pmap),
                _full((cin, 1)), _full((cin, 1)), _full((cin, 1)),
                _full((cout, cin)), _full((cout, 1)),
            ],
            out_specs=[
                pl.BlockSpec((1, cout, tn), pmap),
                _full((1, cout)),
            ],
            out_shape=[
                jax.ShapeDtypeStruct((BB_, cout, N1), f32),
                jax.ShapeDtypeStruct((1, cout), f32),
            ],
        )(pre, m[:, None], r[:, None], bb[:, None], W, wb[:, None])

    pre2, s2 = layer(pre1, s1, BG1, BB1, MW2, Mb2, 512, _NT)
    pre3, s3 = layer(pre2, s2, BG2, BB2, MW3, Mb3, 128, 1)
    pre4, s4 = layer(pre3, s3, BG3, BB3, MW4, Mb4, 128, 1)

    m4 = s4[0] / _CNT
    v4 = _var_pass(pre4, m4[:, None], 128)
    r4 = BG4 / jnp.sqrt(v4 + _EPS)
    mask, oh = pl.pallas_call(
        _k5,
        grid=(BB_,),
        in_specs=[
            pl.BlockSpec((1, 128, N1), lambda b: (b, 0, 0)),
            _full((128, 1)), _full((128, 1)), _full((128, 1)),
            _full((1, 128)), _full((1, 1)),
        ],
        out_specs=[
            pl.BlockSpec((1, 1, N1), lambda b: (b, 0, 0)),
            pl.BlockSpec((1, 1, N1), lambda b: (b, 0, 0)),
        ],
        out_shape=[
            jax.ShapeDtypeStruct((BB_, 1, N1), f32),
            jax.ShapeDtypeStruct((BB_, 1, N1), jnp.int32),
        ],
    )(pre4, m4[:, None], r4[:, None], BB4[:, None], MW5, Mb5[:, None])

    return (mask.reshape(BB_, N1), oh.reshape(BB_, N1),
            se, te)


# speedup vs baseline: 1.1796x; 1.1796x over previous
import jax
import jax.numpy as jnp
from jax.experimental import pallas as pl

BB_ = 16
N1 = 2048
NSUB = 1024
EMB = 512
HID = 256
_EPS = 1e-5
_CNT = BB_ * N1
_NT = 2
_TN = N1 // _NT


def _lrelu(x):
    return jnp.where(x >= 0, x, x * 0.01)


def _mm(a, b):
    return jax.lax.dot_general(a, b, (((1,), (0,)), ((), ())),
                               preferred_element_type=jnp.float32)


def _k1a(src_ref, tgt_ref, W1_ref, be1_ref, W2_ref, be2_ref,
         W1t_ref, bt1_ref, W2t_ref, bt2_ref, se_ref, te_ref):
    sf = _lrelu(_mm(W1_ref[...], src_ref[0]) + be1_ref[...])
    se_ref[0] = _mm(W2_ref[...], sf) + be2_ref[...]
    ia = jnp.max(sf, axis=1, keepdims=True)
    tf = _lrelu(_mm(W1t_ref[...], tgt_ref[0]) + bt1_ref[...])
    te_ref[0] = _mm(W2t_ref[...], tf + ia) + bt2_ref[...]


def _k1b(se_ref, te_ref, MW1_ref, Mb1_ref, pre1_ref, s1_ref, q1_ref):
    b = pl.program_id(0)
    t = pl.program_id(1)
    se = se_ref[0]
    te = te_ref[0]
    sinv = 1.0 / jnp.sqrt(jnp.sum(se * se, axis=0, keepdims=True))
    tinv = 1.0 / jnp.sqrt(jnp.sum(te * te, axis=0, keepdims=True))
    sn = se * sinv
    tn = te * tinv
    cos = jax.lax.dot_general(tn, sn, (((0,), (0,)), ((), ())),
                              preferred_element_type=jnp.float32)
    pre1 = _mm(MW1_ref[...], cos) + Mb1_ref[...]
    pre1_ref[0] = pre1
    s = jnp.sum(pre1, axis=1)[None, :]
    q = jnp.sum(pre1 * pre1, axis=1)[None, :]

    @pl.when((b == 0) & (t == 0))
    def _():
        s1_ref[...] = s
        q1_ref[...] = q

    @pl.when((b != 0) | (t != 0))
    def _():
        s1_ref[...] += s
        q1_ref[...] += q


def _klayer(pre_ref, m_ref, r_ref, bb_ref, W_ref, b_ref, out_ref, s_ref, q_ref):
    i = pl.program_id(0)
    first = i == 0
    if pre_ref.shape[2] != N1:
        t = pl.program_id(1)
        first = first & (t == 0)
    h = _lrelu((pre_ref[0] - m_ref[...]) * r_ref[...] + bb_ref[...])
    pre = _mm(W_ref[...], h) + b_ref[...]
    out_ref[0] = pre
    s = jnp.sum(pre, axis=1)[None, :]
    q = jnp.sum(pre * pre, axis=1)[None, :]

    @pl.when(first)
    def _():
        s_ref[...] = s
        q_ref[...] = q

    @pl.when(jnp.logical_not(first))
    def _():
        s_ref[...] += s
        q_ref[...] += q


def _k5(pre_ref, m_ref, r_ref, bb_ref, W5_ref, b5_ref, mask_ref, oh_ref):
    h = _lrelu((pre_ref[0] - m_ref[...]) * r_ref[...] + bb_ref[...])
    pre5 = _mm(W5_ref[...], h) + b5_ref[...]
    m = jax.nn.sigmoid(pre5)
    mask_ref[0] = m
    vt = jnp.transpose(m)
    col = jax.lax.broadcasted_iota(jnp.int32, (N1, N1), 1)
    row = jax.lax.broadcasted_iota(jnp.int32, (N1, N1), 0)
    greater = (vt > m).astype(jnp.int32)
    eq_lower = ((vt == m) & (row < col)).astype(jnp.int32)
    rank = jnp.sum(greater + eq_lower, axis=0, keepdims=True)
    oh_ref[0] = (rank < NSUB).astype(jnp.int32)


def _full(shape):
    nd = len(shape)
    return pl.BlockSpec(shape, lambda *_: (0,) * nd)


def _stats(s, q):
    m = s[0] / _CNT
    v = q[0] / _CNT - m * m
    return m, v


def kernel(src, tgt, W1, be1, W2, be2, W1t, bt1, W2t, bt2,
           MW1, Mb1, MW2, Mb2, MW3, Mb3, MW4, Mb4, MW5, Mb5,
           BG1, BB1, BG2, BB2, BG3, BB3, BG4, BB4):
    f32 = jnp.float32
    se, te = pl.pallas_call(
        _k1a,
        grid=(BB_,),
        in_specs=[
            pl.BlockSpec((1, 3, N1), lambda b: (b, 0, 0)),
            pl.BlockSpec((1, 3, NSUB), lambda b: (b, 0, 0)),
            _full((HID, 3)), _full((HID, 1)),
            _full((EMB, HID)), _full((EMB, 1)),
            _full((HID, 3)), _full((HID, 1)),
            _full((EMB, HID)), _full((EMB, 1)),
        ],
        out_specs=[
            pl.BlockSpec((1, EMB, N1), lambda b: (b, 0, 0)),
            pl.BlockSpec((1, EMB, NSUB), lambda b: (b, 0, 0)),
        ],
        out_shape=[
            jax.ShapeDtypeStruct((BB_, EMB, N1), f32),
            jax.ShapeDtypeStruct((BB_, EMB, NSUB), f32),
        ],
    )(src, tgt, W1, be1[:, None], W2, be2[:, None],
      W1t, bt1[:, None], W2t, bt2[:, None])

    pre1, s1, q1 = pl.pallas_call(
        _k1b,
        grid=(BB_, _NT),
        in_specs=[
            pl.BlockSpec((1, EMB, _TN), lambda b, t: (b, 0, t)),
            pl.BlockSpec((1, EMB, NSUB), lambda b, t: (b, 0, 0)),
            _full((NSUB, NSUB)), _full((NSUB, 1)),
        ],
        out_specs=[
            pl.BlockSpec((1, NSUB, _TN), lambda b, t: (b, 0, t)),
            _full((1, NSUB)), _full((1, NSUB)),
        ],
        out_shape=[
            jax.ShapeDtypeStruct((BB_, NSUB, N1), f32),
            jax.ShapeDtypeStruct((1, NSUB), f32),
            jax.ShapeDtypeStruct((1, NSUB), f32),
        ],
    )(se, te, MW1, Mb1[:, None])

    def layer(pre, s, q, g, bb, W, wb, cout, nt):
        cin = pre.shape[1]
        tn = N1 // nt
        m, v = _stats(s, q)
        r = g / jnp.sqrt(v + _EPS)
        grid = (BB_, nt) if nt > 1 else (BB_,)
        if nt > 1:
            pmap = lambda b, t: (b, 0, t)
        else:
            pmap = lambda b: (b, 0, 0)
        return pl.pallas_call(
            _klayer,
            grid=grid,
            in_specs=[
                pl.BlockSpec((1, cin, tn), pmap),
                _full((cin, 1)), _full((cin, 1)), _full((cin, 1)),
                _full((cout, cin)), _full((cout, 1)),
            ],
            out_specs=[
                pl.BlockSpec((1, cout, tn), pmap),
                _full((1, cout)), _full((1, cout)),
            ],
            out_shape=[
                jax.ShapeDtypeStruct((BB_, cout, N1), f32),
                jax.ShapeDtypeStruct((1, cout), f32),
                jax.ShapeDtypeStruct((1, cout), f32),
            ],
        )(pre, m[:, None], r[:, None], bb[:, None], W, wb[:, None])

    pre2, s2, q2 = layer(pre1, s1, q1, BG1, BB1, MW2, Mb2, 512, _NT)
    pre3, s3, q3 = layer(pre2, s2, q2, BG2, BB2, MW3, Mb3, 128, 1)
    pre4, s4, q4 = layer(pre3, s3, q3, BG3, BB3, MW4, Mb4, 128, 1)

    m4, v4 = _stats(s4, q4)
    r4 = BG4 / jnp.sqrt(v4 + _EPS)
    mask, oh = pl.pallas_call(
        _k5,
        grid=(BB_,),
        in_specs=[
            pl.BlockSpec((1, 128, N1), lambda b: (b, 0, 0)),
            _full((128, 1)), _full((128, 1)), _full((128, 1)),
            _full((1, 128)), _full((1, 1)),
        ],
        out_specs=[
            pl.BlockSpec((1, 1, N1), lambda b: (b, 0, 0)),
            pl.BlockSpec((1, 1, N1), lambda b: (b, 0, 0)),
        ],
        out_shape=[
            jax.ShapeDtypeStruct((BB_, 1, N1), f32),
            jax.ShapeDtypeStruct((BB_, 1, N1), jnp.int32),
        ],
    )(pre4, m4[:, None], r4[:, None], BB4[:, None], MW5, Mb5[:, None])

    return (mask.reshape(BB_, N1), oh.reshape(BB_, N1),
            se, te)
